# baseline (speedup 1.0000x reference)
import functools

import jax
import jax.numpy as jnp
from jax import lax
from jax.experimental import pallas as pl
from jax.experimental.pallas import tpu as pltpu

N_ROWS = 16384
N_COLS = 1000
BLOCK_ROWS = 2048
GRID = N_ROWS // BLOCK_ROWS
K = max(1, N_ROWS * 10 // 100)


def _fused_kernel(x_ref, t_ref, out_ref, ce_s):
    b = pl.program_id(0)
    x = x_ref[...]
    t = t_ref[0]
    tcol = t.reshape(BLOCK_ROWS, 1)
    m = jnp.max(x, axis=1, keepdims=True)
    s = jnp.sum(jnp.exp(x - m), axis=1, keepdims=True)
    col = lax.broadcasted_iota(jnp.int32, x.shape, 1)
    picked = jnp.sum(jnp.where(col == tcol, x, 0.0), axis=1, keepdims=True)
    ce = (m - picked) + jnp.log(s)
    ce_s[pl.ds(b, 1), :] = ce.reshape(1, BLOCK_ROWS)

    @pl.when(b == GRID - 1)
    def _select():
        cev = ce_s[...]
        bits = lax.bitcast_convert_type(cev, jnp.int32)
        kf = jnp.float32(K)

        def body(_, carry):
            lo, hi = carry
            mid = lo + (hi - lo) // 2
            cnt = jnp.sum((bits >= mid).astype(jnp.int32))
            ge = cnt >= K
            return jnp.where(ge, mid, lo), jnp.where(ge, hi, mid)

        lo0 = jnp.int32(-1)
        hi0 = jnp.int32(0x7F800001)
        lo, _ = lax.fori_loop(0, 31, body, (lo0, hi0))

        gt = bits > lo
        cnt_gt = jnp.sum(gt.astype(jnp.float32))
        sum_gt = jnp.sum(jnp.where(gt, cev, 0.0))
        kth = jnp.max(jnp.where(bits == lo, cev, 0.0))
        out_ref[0, 0] = (sum_gt + kth * (kf - cnt_gt)) / kf


@functools.partial(jax.jit)
def kernel(inputs, targets):
    t3d = targets.astype(jnp.int32).reshape(GRID, 1, BLOCK_ROWS)
    out = pl.pallas_call(
        _fused_kernel,
        grid=(GRID,),
        in_specs=[
            pl.BlockSpec((BLOCK_ROWS, N_COLS), lambda b: (b, 0)),
            pl.BlockSpec((1, 1, BLOCK_ROWS), lambda b: (b, 0, 0)),
        ],
        out_specs=pl.BlockSpec(memory_space=pltpu.SMEM),
        out_shape=jax.ShapeDtypeStruct((1, 1), jnp.float32),
        scratch_shapes=[pltpu.VMEM((GRID, BLOCK_ROWS), jnp.float32)],
    )(inputs, t3d)
    return out.reshape(())


# speedup vs baseline: 1.0066x; 1.0066x over previous
import functools

import jax
import jax.numpy as jnp
from jax import lax
from jax.experimental import pallas as pl
from jax.experimental.pallas import tpu as pltpu

N_ROWS = 16384
N_COLS = 1000
BR = 256
NCHUNK = N_ROWS // BR
NBUF = 8
K = max(1, N_ROWS * 10 // 100)


def _ce_chunk(xbuf, tbuf):
    x = xbuf[...]
    tcol = tbuf[0].reshape(BR, 1)
    m = jnp.max(x, axis=1, keepdims=True)
    s = jnp.sum(jnp.exp(x - m), axis=1, keepdims=True)
    col = lax.broadcasted_iota(jnp.int32, x.shape, 1)
    picked = jnp.sum(jnp.where(col == tcol, x, 0.0), axis=1, keepdims=True)
    return ((m - picked) + jnp.log(s)).reshape(1, BR)


def _fused_kernel(x_hbm, t_hbm, out_ref, *scratch):
    xbufs = scratch[:NBUF]
    tbufs = scratch[NBUF:2 * NBUF]
    sem = scratch[2 * NBUF]
    semt = scratch[2 * NBUF + 1]
    ce_s = scratch[2 * NBUF + 2]

    def start(c, b):
        pltpu.make_async_copy(
            x_hbm.at[pl.ds(c * BR, BR), :], xbufs[b], sem.at[b]).start()
        pltpu.make_async_copy(t_hbm.at[c], tbufs[b], semt.at[b]).start()

    def wait(b):
        pltpu.make_async_copy(
            x_hbm.at[pl.ds(0, BR), :], xbufs[b], sem.at[b]).wait()
        pltpu.make_async_copy(t_hbm.at[0], tbufs[b], semt.at[b]).wait()

    for b in range(NBUF):
        start(b, b)

    def outer(o, _):
        base = o * NBUF
        for b in range(NBUF):
            wait(b)
            ce_s[pl.ds(base + b, 1), :] = _ce_chunk(xbufs[b], tbufs[b])

            @pl.when(base + b + NBUF < NCHUNK)
            def _(b=b):
                start(base + b + NBUF, b)
        return 0

    lax.fori_loop(0, NCHUNK // NBUF, outer, 0, unroll=False)

    cev = ce_s[...]
    bits = lax.bitcast_convert_type(cev, jnp.int32)
    kf = jnp.float32(K)

    def body(_, carry):
        lo, hi = carry
        mid = lo + (hi - lo) // 2
        cnt = jnp.sum((bits >= mid).astype(jnp.int32))
        ge = cnt >= K
        return jnp.where(ge, mid, lo), jnp.where(ge, hi, mid)

    lo0 = jnp.int32(-1)
    hi0 = jnp.int32(0x7F800001)
    lo, _ = lax.fori_loop(0, 31, body, (lo0, hi0))

    gt = bits > lo
    cnt_gt = jnp.sum(gt.astype(jnp.float32))
    sum_gt = jnp.sum(jnp.where(gt, cev, 0.0))
    kth = jnp.max(jnp.where(bits == lo, cev, 0.0))
    out_ref[0, 0] = (sum_gt + kth * (kf - cnt_gt)) / kf


@functools.partial(jax.jit)
def kernel(inputs, targets):
    t32 = targets if targets.dtype == jnp.int32 else targets.astype(jnp.int32)
    t3d = t32.reshape(NCHUNK, 1, BR)
    out = pl.pallas_call(
        _fused_kernel,
        in_specs=[
            pl.BlockSpec(memory_space=pltpu.MemorySpace.HBM),
            pl.BlockSpec(memory_space=pltpu.MemorySpace.HBM),
        ],
        out_specs=pl.BlockSpec(memory_space=pltpu.SMEM),
        out_shape=jax.ShapeDtypeStruct((1, 1), jnp.float32),
        scratch_shapes=(
            [pltpu.VMEM((BR, N_COLS), jnp.float32) for _ in range(NBUF)]
            + [pltpu.VMEM((1, BR), jnp.int32) for _ in range(NBUF)]
            + [pltpu.SemaphoreType.DMA((NBUF,)),
               pltpu.SemaphoreType.DMA((NBUF,)),
               pltpu.VMEM((NCHUNK, BR), jnp.float32)]
        ),
    )(inputs, t3d)
    return out.reshape(())
